# baseline (speedup 1.0000x reference)
import functools

import jax
import jax.numpy as jnp
from jax import lax
from jax.experimental import pallas as pl
from jax.experimental.pallas import tpu as pltpu
from jax.experimental.pallas import tpu_sc as plsc

B, N, D = 8, 4096, 4
BM = 256
RB = N // BM

_NC, _NS, _L = 2, 16, 16
_NW = _NC * _NS
_C = N // _NW
_OW = 48


def _argmin_body(p_ref, t_ref, nnt_ref, nnp_ref, colm_ref, cola_ref):
    rb = pl.program_id(1)
    p = p_ref[0]
    px = p[:, 0:1]
    py = p[:, 1:2]
    tx = t_ref[0, 0:1, :]
    ty = t_ref[0, 1:2, :]
    dx = px - tx
    dy = py - ty
    d2 = dx * dx + dy * dy

    rarg = jnp.argmin(d2, axis=1).astype(jnp.int32)
    nnt_ref[0, 0] = rarg

    bcm = jnp.min(d2, axis=0, keepdims=True)
    bca = (jnp.argmin(d2, axis=0).astype(jnp.int32)[None, :]
           + rb * BM)

    @pl.when(rb == 0)
    def _():
        colm_ref[...] = bcm
        cola_ref[...] = bca

    @pl.when(rb > 0)
    def _():
        old_m = colm_ref[...]
        old_a = cola_ref[...]
        take_new = bcm < old_m
        colm_ref[...] = jnp.where(take_new, bcm, old_m)
        cola_ref[...] = jnp.where(take_new, bca, old_a)

    @pl.when(rb == RB - 1)
    def _():
        nnp_ref[0] = cola_ref[...]


def _argmins(preds, targs_t8, interpret=False):
    return pl.pallas_call(
        _argmin_body,
        grid=(B, RB),
        in_specs=[
            pl.BlockSpec((1, BM, D), lambda b, rb: (b, rb, 0)),
            pl.BlockSpec((1, 8, N), lambda b, rb: (b, 0, 0)),
        ],
        out_specs=[
            pl.BlockSpec((1, 1, BM), lambda b, rb: (b * RB + rb, 0, 0)),
            pl.BlockSpec((1, 1, N), lambda b, rb: (b, 0, 0)),
        ],
        out_shape=[
            jax.ShapeDtypeStruct((B * RB, 1, BM), jnp.int32),
            jax.ShapeDtypeStruct((B, 1, N), jnp.int32),
        ],
        scratch_shapes=[
            pltpu.VMEM((1, N), jnp.float32),
            pltpu.VMEM((1, N), jnp.int32),
        ],
        compiler_params=pltpu.CompilerParams(
            dimension_semantics=("arbitrary", "arbitrary"),
        ),
        interpret=interpret,
    )(preds, targs_t8)


def _sc_body(px_hbm, py_hbm, tx_hbm, ty_hbm, nnt_hbm, nnp_hbm, out_hbm,
             v0, v1, idxv, cxv, cyv, stage):
    wid = lax.axis_index("c") * _NS + lax.axis_index("s")
    base = wid * _C
    accx = jnp.zeros((_L,), jnp.float32)
    accy = jnp.zeros((_L,), jnp.float32)
    acct = jnp.zeros((_L,), jnp.float32)

    pltpu.sync_copy(tx_hbm, v0)
    pltpu.sync_copy(ty_hbm, v1)
    for b in range(B):
        pltpu.sync_copy(nnt_hbm.at[pl.ds(b * N + base, _C)], idxv)
        pltpu.sync_copy(px_hbm.at[pl.ds(b * N + base, _C)], cxv)
        pltpu.sync_copy(py_hbm.at[pl.ds(b * N + base, _C)], cyv)
        for v in range(_C // _L):
            it = idxv[pl.ds(v * _L, _L)] + jnp.int32(b * N)
            gx = plsc.load_gather(v0, [it])
            gy = plsc.load_gather(v1, [it])
            accx = accx + jnp.abs(cxv[pl.ds(v * _L, _L)] - gx)
            accy = accy + jnp.abs(cyv[pl.ds(v * _L, _L)] - gy)

    pltpu.sync_copy(px_hbm, v0)
    pltpu.sync_copy(py_hbm, v1)
    for b in range(B):
        pltpu.sync_copy(nnp_hbm.at[pl.ds(b * N + base, _C)], idxv)
        pltpu.sync_copy(tx_hbm.at[pl.ds(b * N + base, _C)], cxv)
        pltpu.sync_copy(ty_hbm.at[pl.ds(b * N + base, _C)], cyv)
        for v in range(_C // _L):
            ip = idxv[pl.ds(v * _L, _L)] + jnp.int32(b * N)
            gx = plsc.load_gather(v0, [ip])
            gy = plsc.load_gather(v1, [ip])
            acct = (acct + jnp.abs(gx - cxv[pl.ds(v * _L, _L)])
                    + jnp.abs(gy - cyv[pl.ds(v * _L, _L)]))

    stage[pl.ds(0, _L)] = accx
    stage[pl.ds(_L, _L)] = accy
    stage[pl.ds(2 * _L, _L)] = acct
    pltpu.sync_copy(stage, out_hbm.at[pl.ds(wid * _OW, _OW)])


@functools.partial(jax.jit, static_argnames=())
def _sc_gather_l1(px, py, tx, ty, nnt, nnp):
    mesh = plsc.VectorSubcoreMesh(core_axis_name="c", subcore_axis_name="s")
    return pl.kernel(
        _sc_body,
        out_type=jax.ShapeDtypeStruct((_NW * _OW,), jnp.float32),
        mesh=mesh,
        scratch_types=[
            pltpu.VMEM((B * N,), jnp.float32),
            pltpu.VMEM((B * N,), jnp.float32),
            pltpu.VMEM((_C,), jnp.int32),
            pltpu.VMEM((_C,), jnp.float32),
            pltpu.VMEM((_C,), jnp.float32),
            pltpu.VMEM((_OW,), jnp.float32),
        ],
        compiler_params=pltpu.CompilerParams(needs_layout_passes=False),
    )(px, py, tx, ty, nnt, nnp)


def kernel(preds, targs, subcoef):
    tt = jnp.transpose(targs, (0, 2, 1))
    tt8 = jnp.concatenate(
        [tt, jnp.zeros((B, 8 - D, N), jnp.float32)], axis=1)
    nnt, nnp = _argmins(preds, tt8)
    nnt = nnt.reshape(B * N)
    nnp = nnp.reshape(B * N)

    px = preds[:, :, 0].reshape(B * N)
    py = preds[:, :, 1].reshape(B * N)
    tx = targs[:, :, 0].reshape(B * N)
    ty = targs[:, :, 1].reshape(B * N)
    parts = _sc_gather_l1(px, py, tx, ty, nnt, nnp).reshape(_NW, 3, _L)
    sums = jnp.sum(parts, axis=(0, 2))
    return subcoef[0] * sums[0] + subcoef[1] * sums[1] + sums[2]


# speedup vs baseline: 1.5152x; 1.5152x over previous
import functools

import jax
import jax.numpy as jnp
from jax import lax
from jax.experimental import pallas as pl
from jax.experimental.pallas import tpu as pltpu
from jax.experimental.pallas import tpu_sc as plsc

B, N, D = 8, 4096, 4
BM = 256
RB = N // BM

_NC, _NS, _L = 2, 16, 16
_NW = _NC * _NS
_C = N // _NW
_OW = 48


def _argmin_body(lhs_ref, rhsa_ref, rhsb_ref, nnt_ref, nnp_ref,
                 colm_ref, cola_ref):
    rb = pl.program_id(1)
    lhs = lhs_ref[0]
    rhsa = rhsa_ref[0]
    rhsb = rhsb_ref[0]
    dn = (((1,), (0,)), ((), ()))
    rowkey = lax.dot_general(lhs, rhsa, dn,
                             preferred_element_type=jnp.float32)
    colkey = lax.dot_general(lhs, rhsb, dn,
                             preferred_element_type=jnp.float32)

    rarg = jnp.argmin(rowkey, axis=1).astype(jnp.int32)
    nnt_ref[0, 0] = rarg

    bcm = jnp.min(colkey, axis=0, keepdims=True)
    bca = (jnp.argmin(colkey, axis=0).astype(jnp.int32)[None, :]
           + rb * BM)

    @pl.when(rb == 0)
    def _():
        colm_ref[...] = bcm
        cola_ref[...] = bca

    @pl.when(rb > 0)
    def _():
        old_m = colm_ref[...]
        old_a = cola_ref[...]
        take_new = bcm < old_m
        colm_ref[...] = jnp.where(take_new, bcm, old_m)
        cola_ref[...] = jnp.where(take_new, bca, old_a)

    @pl.when(rb == RB - 1)
    def _():
        nnp_ref[0] = cola_ref[...]


def _argmins(lhs, rhsa, rhsb, interpret=False):
    return pl.pallas_call(
        _argmin_body,
        grid=(B, RB),
        in_specs=[
            pl.BlockSpec((1, BM, 8), lambda b, rb: (b, rb, 0)),
            pl.BlockSpec((1, 8, N), lambda b, rb: (b, 0, 0)),
            pl.BlockSpec((1, 8, N), lambda b, rb: (b, 0, 0)),
        ],
        out_specs=[
            pl.BlockSpec((1, 1, BM), lambda b, rb: (b * RB + rb, 0, 0)),
            pl.BlockSpec((1, 1, N), lambda b, rb: (b, 0, 0)),
        ],
        out_shape=[
            jax.ShapeDtypeStruct((B * RB, 1, BM), jnp.int32),
            jax.ShapeDtypeStruct((B, 1, N), jnp.int32),
        ],
        scratch_shapes=[
            pltpu.VMEM((1, N), jnp.float32),
            pltpu.VMEM((1, N), jnp.int32),
        ],
        compiler_params=pltpu.CompilerParams(
            dimension_semantics=("arbitrary", "arbitrary"),
        ),
        interpret=interpret,
    )(lhs, rhsa, rhsb)


def _sc_body(px_hbm, py_hbm, tx_hbm, ty_hbm, nnt_hbm, nnp_hbm, out_hbm,
             v0, v1, idxv, cxv, cyv, stage):
    wid = lax.axis_index("c") * _NS + lax.axis_index("s")
    base = wid * _C
    accx = jnp.zeros((_L,), jnp.float32)
    accy = jnp.zeros((_L,), jnp.float32)
    acct = jnp.zeros((_L,), jnp.float32)

    pltpu.sync_copy(tx_hbm, v0)
    pltpu.sync_copy(ty_hbm, v1)
    for b in range(B):
        pltpu.sync_copy(nnt_hbm.at[pl.ds(b * N + base, _C)], idxv)
        pltpu.sync_copy(px_hbm.at[pl.ds(b * N + base, _C)], cxv)
        pltpu.sync_copy(py_hbm.at[pl.ds(b * N + base, _C)], cyv)
        for v in range(_C // _L):
            it = idxv[pl.ds(v * _L, _L)] + jnp.int32(b * N)
            gx = plsc.load_gather(v0, [it])
            gy = plsc.load_gather(v1, [it])
            accx = accx + jnp.abs(cxv[pl.ds(v * _L, _L)] - gx)
            accy = accy + jnp.abs(cyv[pl.ds(v * _L, _L)] - gy)

    pltpu.sync_copy(px_hbm, v0)
    pltpu.sync_copy(py_hbm, v1)
    for b in range(B):
        pltpu.sync_copy(nnp_hbm.at[pl.ds(b * N + base, _C)], idxv)
        pltpu.sync_copy(tx_hbm.at[pl.ds(b * N + base, _C)], cxv)
        pltpu.sync_copy(ty_hbm.at[pl.ds(b * N + base, _C)], cyv)
        for v in range(_C // _L):
            ip = idxv[pl.ds(v * _L, _L)] + jnp.int32(b * N)
            gx = plsc.load_gather(v0, [ip])
            gy = plsc.load_gather(v1, [ip])
            acct = (acct + jnp.abs(gx - cxv[pl.ds(v * _L, _L)])
                    + jnp.abs(gy - cyv[pl.ds(v * _L, _L)]))

    stage[pl.ds(0, _L)] = accx
    stage[pl.ds(_L, _L)] = accy
    stage[pl.ds(2 * _L, _L)] = acct
    pltpu.sync_copy(stage, out_hbm.at[pl.ds(wid * _OW, _OW)])


@functools.partial(jax.jit, static_argnames=())
def _sc_gather_l1(px, py, tx, ty, nnt, nnp):
    mesh = plsc.VectorSubcoreMesh(core_axis_name="c", subcore_axis_name="s")
    return pl.kernel(
        _sc_body,
        out_type=jax.ShapeDtypeStruct((_NW * _OW,), jnp.float32),
        mesh=mesh,
        scratch_types=[
            pltpu.VMEM((B * N,), jnp.float32),
            pltpu.VMEM((B * N,), jnp.float32),
            pltpu.VMEM((_C,), jnp.int32),
            pltpu.VMEM((_C,), jnp.float32),
            pltpu.VMEM((_C,), jnp.float32),
            pltpu.VMEM((_OW,), jnp.float32),
        ],
        compiler_params=pltpu.CompilerParams(needs_layout_passes=False),
    )(px, py, tx, ty, nnt, nnp)


def kernel(preds, targs, subcoef):
    pxy = preds[:, :, :2]
    p2 = jnp.sum(pxy * pxy, axis=-1, keepdims=True)
    ones = jnp.ones((B, N, 1), jnp.float32)
    zeros = jnp.zeros((B, N, 4), jnp.float32)
    lhs = jnp.concatenate([pxy, ones, p2, zeros], axis=-1)

    txy = targs[:, :, :2]
    t2 = jnp.sum(txy * txy, axis=-1, keepdims=True)
    zt = jnp.zeros((B, N, 1), jnp.float32)
    rhsa = jnp.transpose(
        jnp.concatenate([-2.0 * txy, t2, zt, zeros], axis=-1), (0, 2, 1))
    rhsb = jnp.transpose(
        jnp.concatenate([-2.0 * txy, zt, ones, zeros], axis=-1), (0, 2, 1))

    nnt, nnp = _argmins(lhs, rhsa, rhsb)
    nnt = nnt.reshape(B * N)
    nnp = nnp.reshape(B * N)

    px = preds[:, :, 0].reshape(B * N)
    py = preds[:, :, 1].reshape(B * N)
    tx = targs[:, :, 0].reshape(B * N)
    ty = targs[:, :, 1].reshape(B * N)
    parts = _sc_gather_l1(px, py, tx, ty, nnt, nnp).reshape(_NW, 3, _L)
    sums = jnp.sum(parts, axis=(0, 2))
    return subcoef[0] * sums[0] + subcoef[1] * sums[1] + sums[2]
